# Initial kernel scaffold; baseline (speedup 1.0000x reference)
import jax
import jax.numpy as jnp
from jax.experimental import pallas as pl


def kernel(z, dummy1, batch, dummy2, dummy3, dummy4, W1, b1, W2, b2):
    raise NotImplementedError("write your pallas kernel here")


# speedup vs baseline: 15.3542x; 15.3542x over previous
import jax
import jax.numpy as jnp
from jax import lax
from jax.experimental import pallas as pl
from jax.experimental.pallas import tpu as pltpu
from jax.experimental.pallas import tpu_sc as plsc

E = 128
B_SEG = 1024
N_NODES = 100000
NC, NS, L = 2, 16, 16
NW = NC * NS
C = 3128
NCHUNK = 25
CPAD = NCHUNK * 128
KEYS = B_SEG * E
DUMMY = KEYS
HIST_PAD = KEYS + 256
STRIPE0 = HIST_PAD // NS
STRIPE1 = KEYS // NS


def _sc_hist_body(z_hbm, batch_hbm, out_hbm, zb_v, bb_v, keys_v, ones_v,
                  zero_v, hist_sh):
    cid = lax.axis_index("c")
    sid = lax.axis_index("s")
    wid = cid * NS + sid
    lo = wid * C
    hi = jnp.minimum(lo + C, N_NODES)
    g0 = jnp.minimum(lo, N_NODES - C)

    pltpu.sync_copy(z_hbm.at[pl.ds(g0, C)], zb_v.at[pl.ds(0, C)])
    pltpu.sync_copy(batch_hbm.at[pl.ds(g0, C)], bb_v.at[pl.ds(0, C)])

    for j in range(8):
        ones_v[pl.ds(j * L, L)] = jnp.full((L,), 1.0, jnp.float32)

    def _zero(i, _):
        zero_v[pl.ds(i * L, L)] = jnp.zeros((L,), jnp.float32)
        return 0
    lax.fori_loop(0, STRIPE0 // L, _zero, 0)

    iota = lax.broadcasted_iota(jnp.int32, (L,), 0)

    def _keys(r, _):
        base = r * 128
        for j in range(8):
            off = base + j * L
            bb = bb_v[pl.ds(off, L)]
            zz = zb_v[pl.ds(off, L)]
            gi = g0 + off + iota
            valid = (gi >= lo) & (gi < hi)
            key = jnp.where(valid, bb * E + zz, DUMMY)
            keys_v[r, pl.ds(j * L, L)] = key
        return 0
    lax.fori_loop(0, NCHUNK, _keys, 0)

    pltpu.sync_copy(zero_v, hist_sh.at[pl.ds(sid * STRIPE0, STRIPE0)])
    plsc.subcore_barrier()

    def _scat(r, _):
        pltpu.sync_copy(ones_v, hist_sh.at[keys_v.at[r]], add=True)
        return 0
    lax.fori_loop(0, NCHUNK, _scat, 0)
    plsc.subcore_barrier()

    pltpu.sync_copy(hist_sh.at[pl.ds(sid * STRIPE1, STRIPE1)],
                    out_hbm.at[cid, pl.ds(sid * STRIPE1, STRIPE1)])


_sc_hist = pl.kernel(
    _sc_hist_body,
    out_type=jax.ShapeDtypeStruct((NC, KEYS), jnp.float32),
    mesh=plsc.VectorSubcoreMesh(core_axis_name="c", subcore_axis_name="s",
                                num_cores=NC, num_subcores=NS),
    scratch_types=[
        pltpu.VMEM((CPAD,), jnp.int32),
        pltpu.VMEM((CPAD,), jnp.int32),
        pltpu.VMEM((NCHUNK, 128), jnp.int32),
        pltpu.VMEM((128,), jnp.float32),
        pltpu.VMEM((STRIPE0,), jnp.float32),
        pltpu.VMEM_SHARED((HIST_PAD,), jnp.float32),
    ],
)


def _mlp_body(p_ref, w1_ref, b1_ref, w2_ref, b2_ref, o_ref):
    h = p_ref[0] + p_ref[1]
    counts = jnp.sum(h, axis=1, keepdims=True)
    x = h / jnp.maximum(counts, 1.0)
    a = jnp.dot(x, w1_ref[...], preferred_element_type=jnp.float32)
    a = jnp.maximum(a + b1_ref[...][None, :], 0.0)
    o_ref[...] = (jnp.dot(a, w2_ref[...], preferred_element_type=jnp.float32)
                  + b2_ref[...][None, :])


_mlp = pl.pallas_call(
    _mlp_body,
    out_shape=jax.ShapeDtypeStruct((B_SEG, 1), jnp.float32),
)


def kernel(z, dummy1, batch, dummy2, dummy3, dummy4, W1, b1, W2, b2):
    part = _sc_hist(z.astype(jnp.int32), batch.astype(jnp.int32))
    return _mlp(part.reshape(NC, B_SEG, E), W1, b1, W2, b2)
